# baseline (speedup 1.0000x reference)
import functools

import jax
import jax.numpy as jnp
from jax import lax
from jax.experimental import pallas as pl
from jax.experimental.pallas import tpu as pltpu
from jax.experimental.pallas import tpu_sc as plsc

BATCH = 4096
HIST = 50
NUM_ITEMS = 10
EMBED_DIM = 128

_INFO = plsc.get_sparse_core_info()
_NC = _INFO.num_cores
_NS = _INFO.num_subcores
_NW = _NC * _NS

_TOTAL = BATCH * HIST
_PER_W = _TOTAL // _NW
_CHUNK = 128
_NCHUNK = _PER_W // _CHUNK


def _sc_lookup(x_blocks, table):
    mesh = plsc.VectorSubcoreMesh(core_axis_name="c", subcore_axis_name="s")

    @functools.partial(
        pl.kernel,
        mesh=mesh,
        out_type=jax.ShapeDtypeStruct((_TOTAL, EMBED_DIM), jnp.float32),
        scratch_types=[
            pltpu.VMEM((_NCHUNK, _CHUNK), jnp.int32),
            pltpu.VMEM((_CHUNK, EMBED_DIM), jnp.float32),
            pltpu.SemaphoreType.DMA,
        ],
    )
    def k(x_hbm, table_hbm, out_hbm, idx_v, rows_v, sem):
        wid = lax.axis_index("s") * _NC + lax.axis_index("c")
        pltpu.sync_copy(x_hbm.at[wid], idx_v)
        base = wid * _PER_W

        def body(j, carry):
            pltpu.async_copy(table_hbm.at[idx_v.at[j]], rows_v, sem).wait()
            pltpu.sync_copy(rows_v, out_hbm.at[pl.ds(base + j * _CHUNK, _CHUNK)])
            return carry

        lax.fori_loop(0, _NCHUNK, body, 0, unroll=False)

    return k(x_blocks, table)


def kernel(x, table):
    ids = x.astype(jnp.int32).reshape(_NW, _NCHUNK, _CHUNK)
    out = _sc_lookup(ids, table)
    return out.reshape(BATCH, HIST, EMBED_DIM)


# speedup vs baseline: 4.2111x; 4.2111x over previous
import functools

import jax
import jax.numpy as jnp
from jax import lax
from jax.experimental import pallas as pl
from jax.experimental.pallas import tpu as pltpu
from jax.experimental.pallas import tpu_sc as plsc

BATCH = 4096
HIST = 50
NUM_ITEMS = 10
EMBED_DIM = 128

_INFO = plsc.get_sparse_core_info()
_NC = _INFO.num_cores
_NS = _INFO.num_subcores
_NW = _NC * _NS

_TOTAL = BATCH * HIST
_PER_W = _TOTAL // _NW
_CHUNK = 128
_K = 5
_NWIN = _PER_W // (_CHUNK * _K)


def _sc_lookup(x_blocks, table):
    mesh = plsc.VectorSubcoreMesh(core_axis_name="c", subcore_axis_name="s")

    @functools.partial(
        pl.kernel,
        mesh=mesh,
        out_type=jax.ShapeDtypeStruct((_TOTAL, EMBED_DIM), jnp.float32),
        scratch_types=[
            pltpu.VMEM_SHARED((NUM_ITEMS, EMBED_DIM), jnp.float32),
            pltpu.VMEM((_PER_W // _CHUNK, _CHUNK), jnp.int32),
            pltpu.VMEM((_K, _CHUNK, EMBED_DIM), jnp.float32),
            pltpu.SemaphoreType.DMA,
            pltpu.SemaphoreType.DMA,
        ],
    )
    def k(x_hbm, table_hbm, out_hbm, tab_sh, idx_v, bufs_v, gsem, ssem):
        sid = lax.axis_index("s")
        wid = sid * _NC + lax.axis_index("c")

        @pl.when(sid == 0)
        def _():
            pltpu.sync_copy(table_hbm, tab_sh)

        pltpu.sync_copy(x_hbm.at[wid], idx_v)
        plsc.subcore_barrier()

        base = wid * _PER_W

        def window(g, carry):
            gh = [
                pltpu.async_copy(
                    tab_sh.at[idx_v.at[g * _K + b]], bufs_v.at[b], gsem
                )
                for b in range(_K)
            ]
            for h in gh:
                h.wait()
            sh = [
                pltpu.async_copy(
                    bufs_v.at[b],
                    out_hbm.at[pl.ds(base + (g * _K + b) * _CHUNK, _CHUNK)],
                    ssem,
                )
                for b in range(_K)
            ]
            for h in sh:
                h.wait()
            return carry

        lax.fori_loop(0, _NWIN, window, 0, unroll=False)

    return k(x_blocks, table)


def kernel(x, table):
    ids = x.astype(jnp.int32).reshape(_NW, _PER_W // _CHUNK, _CHUNK)
    out = _sc_lookup(ids, table)
    return out.reshape(BATCH, HIST, EMBED_DIM)


# speedup vs baseline: 4.2177x; 1.0016x over previous
import functools

import jax
import jax.numpy as jnp
from jax import lax
from jax.experimental import pallas as pl
from jax.experimental.pallas import tpu as pltpu
from jax.experimental.pallas import tpu_sc as plsc

BATCH = 4096
HIST = 50
NUM_ITEMS = 10
EMBED_DIM = 128

_INFO = plsc.get_sparse_core_info()
_NC = _INFO.num_cores
_NS = _INFO.num_subcores
_NW = _NC * _NS

_TOTAL = BATCH * HIST
_PER_W = _TOTAL // _NW
_CHUNK = 128
_K = 5
_NWIN = _PER_W // (_CHUNK * _K)


def _sc_lookup(x_blocks, table):
    mesh = plsc.VectorSubcoreMesh(core_axis_name="c", subcore_axis_name="s")

    @functools.partial(
        pl.kernel,
        mesh=mesh,
        out_type=jax.ShapeDtypeStruct((_TOTAL, EMBED_DIM), jnp.float32),
        scratch_types=[
            pltpu.VMEM_SHARED((_NS * NUM_ITEMS, EMBED_DIM), jnp.float32),
            pltpu.VMEM((_PER_W // _CHUNK, _CHUNK), jnp.int32),
            pltpu.VMEM((_K, _CHUNK, EMBED_DIM), jnp.float32),
            pltpu.SemaphoreType.DMA,
            pltpu.SemaphoreType.DMA,
        ],
    )
    def k(x_hbm, table_hbm, out_hbm, tab_sh, idx_v, bufs_v, gsem, ssem):
        sid = lax.axis_index("s")
        wid = sid * _NC + lax.axis_index("c")

        pltpu.sync_copy(table_hbm, tab_sh.at[pl.ds(sid * NUM_ITEMS, NUM_ITEMS)])
        pltpu.sync_copy(x_hbm.at[wid], idx_v)
        plsc.subcore_barrier()

        base = wid * _PER_W

        def window(g, carry):
            gh = [
                pltpu.async_copy(
                    tab_sh.at[idx_v.at[g * _K + b]], bufs_v.at[b], gsem
                )
                for b in range(_K)
            ]
            for h in gh:
                h.wait()
            sh = [
                pltpu.async_copy(
                    bufs_v.at[b],
                    out_hbm.at[pl.ds(base + (g * _K + b) * _CHUNK, _CHUNK)],
                    ssem,
                )
                for b in range(_K)
            ]
            for h in sh:
                h.wait()
            return carry

        lax.fori_loop(0, _NWIN, window, 0, unroll=False)

    return k(x_blocks, table)


def kernel(x, table):
    ids = x.astype(jnp.int32).reshape(_NW, _PER_W // _CHUNK, _CHUNK)
    sub = (jnp.arange(_NW, dtype=jnp.int32) // _NC) * NUM_ITEMS
    ids = ids + sub[:, None, None]
    out = _sc_lookup(ids, table)
    return out.reshape(BATCH, HIST, EMBED_DIM)


# speedup vs baseline: 4.5357x; 1.0754x over previous
import functools

import jax
import jax.numpy as jnp
from jax import lax
from jax.experimental import pallas as pl
from jax.experimental.pallas import tpu as pltpu
from jax.experimental.pallas import tpu_sc as plsc

BATCH = 4096
HIST = 50
NUM_ITEMS = 10
EMBED_DIM = 128

_INFO = plsc.get_sparse_core_info()
_NC = _INFO.num_cores
_NS = _INFO.num_subcores
_NW = _NC * _NS

_TOTAL = BATCH * HIST
_PER_W = _TOTAL // _NW
_CHUNK = 128
_K = 5
_NWIN = _PER_W // (_CHUNK * _K)


def _sc_lookup(x_blocks, table):
    mesh = plsc.VectorSubcoreMesh(core_axis_name="c", subcore_axis_name="s")

    @functools.partial(
        pl.kernel,
        mesh=mesh,
        out_type=jax.ShapeDtypeStruct((_TOTAL, EMBED_DIM), jnp.float32),
        scratch_types=[
            pltpu.VMEM_SHARED((_NS * NUM_ITEMS, EMBED_DIM), jnp.float32),
            pltpu.VMEM((_PER_W // _CHUNK, _CHUNK), jnp.int32),
            pltpu.VMEM((_K, _CHUNK, EMBED_DIM), jnp.float32),
        ]
        + [pltpu.SemaphoreType.DMA] * (2 * _K),
    )
    def k(x_hbm, table_hbm, out_hbm, tab_sh, idx_v, bufs_v, *sems):
        gsem = sems[:_K]
        ssem = sems[_K:]
        sid = lax.axis_index("s")
        wid = sid * _NC + lax.axis_index("c")

        pltpu.sync_copy(table_hbm, tab_sh.at[pl.ds(sid * NUM_ITEMS, NUM_ITEMS)])
        pltpu.sync_copy(x_hbm.at[wid], idx_v)
        plsc.subcore_barrier()

        base = wid * _PER_W

        def window(g, carry):
            gh = []
            for b in range(_K):
                @pl.when(g > 0)
                def _(b=b):
                    pltpu.make_async_copy(
                        bufs_v.at[b], out_hbm.at[pl.ds(0, _CHUNK)], ssem[b]
                    ).wait()

                gh.append(
                    pltpu.async_copy(
                        tab_sh.at[idx_v.at[g * _K + b]], bufs_v.at[b], gsem[b]
                    )
                )
            for b in range(_K):
                gh[b].wait()
                pltpu.async_copy(
                    bufs_v.at[b],
                    out_hbm.at[pl.ds(base + (g * _K + b) * _CHUNK, _CHUNK)],
                    ssem[b],
                )
            return carry

        lax.fori_loop(0, _NWIN, window, 0, unroll=False)

        for b in range(_K):
            pltpu.make_async_copy(
                bufs_v.at[b], out_hbm.at[pl.ds(0, _CHUNK)], ssem[b]
            ).wait()

    return k(x_blocks, table)


def kernel(x, table):
    ids = x.astype(jnp.int32).reshape(_NW, _PER_W // _CHUNK, _CHUNK)
    sub = (jnp.arange(_NW, dtype=jnp.int32) // _NC) * NUM_ITEMS
    ids = ids + sub[:, None, None]
    out = _sc_lookup(ids, table)
    return out.reshape(BATCH, HIST, EMBED_DIM)


# speedup vs baseline: 7.3931x; 1.6300x over previous
import functools

import jax
import jax.numpy as jnp
from jax import lax
from jax.experimental import pallas as pl
from jax.experimental.pallas import tpu as pltpu
from jax.experimental.pallas import tpu_sc as plsc

BATCH = 4096
HIST = 50
NUM_ITEMS = 10
EMBED_DIM = 128

_INFO = plsc.get_sparse_core_info()
_NC = _INFO.num_cores
_NS = _INFO.num_subcores
_NW = _NC * _NS

_HPAD = 56
_B_PER_W = BATCH // _NW
_CHUNK_B = 2
_CHUNK = _CHUNK_B * _HPAD
_NCHUNK = _B_PER_W // _CHUNK_B
_K = 4
_NWIN = _NCHUNK // _K
_PER_W_PAD = _B_PER_W * _HPAD


def _sc_lookup(x_blocks, table):
    mesh = plsc.VectorSubcoreMesh(core_axis_name="c", subcore_axis_name="s")

    @functools.partial(
        pl.kernel,
        mesh=mesh,
        out_type=jax.ShapeDtypeStruct((BATCH * _HPAD, EMBED_DIM), jnp.float32),
        scratch_types=[
            pltpu.VMEM_SHARED((_NS * NUM_ITEMS, EMBED_DIM), jnp.float32),
            pltpu.VMEM((_NCHUNK, _CHUNK), jnp.int32),
            pltpu.VMEM((_K, _CHUNK, EMBED_DIM), jnp.float32),
        ]
        + [pltpu.SemaphoreType.DMA] * (2 * _K),
    )
    def k(x_hbm, table_hbm, out_hbm, tab_sh, idx_v, bufs_v, *sems):
        gsem = sems[:_K]
        ssem = sems[_K:]
        sid = lax.axis_index("s")
        wid = sid * _NC + lax.axis_index("c")

        pltpu.sync_copy(table_hbm, tab_sh.at[pl.ds(sid * NUM_ITEMS, NUM_ITEMS)])
        pltpu.sync_copy(x_hbm.at[wid], idx_v)
        plsc.subcore_barrier()

        base = wid * _PER_W_PAD

        def window(g, carry):
            gh = []
            for b in range(_K):
                @pl.when(g > 0)
                def _(b=b):
                    pltpu.make_async_copy(
                        bufs_v.at[b], out_hbm.at[pl.ds(0, _CHUNK)], ssem[b]
                    ).wait()

                gh.append(
                    pltpu.async_copy(
                        tab_sh.at[idx_v.at[g * _K + b]], bufs_v.at[b], gsem[b]
                    )
                )
            for b in range(_K):
                gh[b].wait()
                pltpu.async_copy(
                    bufs_v.at[b],
                    out_hbm.at[pl.ds(base + (g * _K + b) * _CHUNK, _CHUNK)],
                    ssem[b],
                )
            return carry

        lax.fori_loop(0, _NWIN, window, 0, unroll=False)

        for b in range(_K):
            pltpu.make_async_copy(
                bufs_v.at[b], out_hbm.at[pl.ds(0, _CHUNK)], ssem[b]
            ).wait()

    return k(x_blocks, table)


def kernel(x, table):
    ids = x.astype(jnp.int32)
    ids = jnp.pad(ids, ((0, 0), (0, _HPAD - HIST)))
    ids = ids.reshape(_NW, _NCHUNK, _CHUNK)
    sub = (jnp.arange(_NW, dtype=jnp.int32) // _NC) * NUM_ITEMS
    ids = ids + sub[:, None, None]
    out = _sc_lookup(ids, table)
    return out.reshape(BATCH, _HPAD, EMBED_DIM)[:, :HIST, :]


# speedup vs baseline: 8.2543x; 1.1165x over previous
import functools

import jax
import jax.numpy as jnp
from jax import lax
from jax.experimental import pallas as pl
from jax.experimental.pallas import tpu as pltpu
from jax.experimental.pallas import tpu_sc as plsc

BATCH = 4096
HIST = 50
NUM_ITEMS = 10
EMBED_DIM = 128

_INFO = plsc.get_sparse_core_info()
_NC = _INFO.num_cores
_NS = _INFO.num_subcores
_NW = _NC * _NS

_HPAD = 56
_B_PER_W = BATCH // _NW
_CHUNK_B = 2
_CHUNK = 128
_NCHUNK = _B_PER_W // _CHUNK_B
_K = 4
_NWIN = _NCHUNK // _K


def _sc_lookup(x_blocks, table):
    mesh = plsc.VectorSubcoreMesh(core_axis_name="c", subcore_axis_name="s")

    @functools.partial(
        pl.kernel,
        mesh=mesh,
        out_type=jax.ShapeDtypeStruct((BATCH, HIST, EMBED_DIM), jnp.float32),
        scratch_types=[
            pltpu.VMEM_SHARED((_NS * NUM_ITEMS, EMBED_DIM), jnp.float32),
            pltpu.VMEM((_NCHUNK, _CHUNK), jnp.int32),
            pltpu.VMEM((_K, _CHUNK, EMBED_DIM), jnp.float32),
        ]
        + [pltpu.SemaphoreType.DMA] * (2 * _K),
        compiler_params=pltpu.CompilerParams(use_tc_tiling_on_sc=True),
    )
    def k(x_hbm, table_hbm, out_hbm, tab_sh, idx_v, bufs_v, *sems):
        gsem = sems[:_K]
        ssem = sems[_K:]
        sid = lax.axis_index("s")
        wid = sid * _NC + lax.axis_index("c")

        pltpu.sync_copy(table_hbm, tab_sh.at[pl.ds(sid * NUM_ITEMS, NUM_ITEMS)])
        pltpu.sync_copy(x_hbm.at[wid], idx_v)
        plsc.subcore_barrier()

        batch0 = wid * _B_PER_W

        def window(g, carry):
            gh = []
            for b in range(_K):
                @pl.when(g > 0)
                def _(b=b):
                    for _i in range(_CHUNK_B):
                        pltpu.make_async_copy(
                            bufs_v.at[b, pl.ds(0, HIST)],
                            out_hbm.at[0],
                            ssem[b],
                        ).wait()

                gh.append(
                    pltpu.async_copy(
                        tab_sh.at[idx_v.at[g * _K + b]], bufs_v.at[b], gsem[b]
                    )
                )
            for b in range(_K):
                gh[b].wait()
                for i in range(_CHUNK_B):
                    pltpu.async_copy(
                        bufs_v.at[b, pl.ds(i * _HPAD, HIST)],
                        out_hbm.at[batch0 + (g * _K + b) * _CHUNK_B + i],
                        ssem[b],
                    )
            return carry

        lax.fori_loop(0, _NWIN, window, 0, unroll=False)

        for b in range(_K):
            for _i in range(_CHUNK_B):
                pltpu.make_async_copy(
                    bufs_v.at[b, pl.ds(0, HIST)],
                    out_hbm.at[0],
                    ssem[b],
                ).wait()

    return k(x_blocks, table)


def kernel(x, table):
    ids = x.astype(jnp.int32)
    ids = jnp.pad(ids, ((0, 0), (0, _HPAD - HIST)))
    ids = ids.reshape(_NW, _NCHUNK, _CHUNK_B * _HPAD)
    ids = jnp.pad(ids, ((0, 0), (0, 0), (0, _CHUNK - _CHUNK_B * _HPAD)))
    sub = (jnp.arange(_NW, dtype=jnp.int32) // _NC) * NUM_ITEMS
    ids = ids + sub[:, None, None]
    return _sc_lookup(ids, table)


# speedup vs baseline: 18.9480x; 2.2955x over previous
import functools

import jax
import jax.numpy as jnp
from jax import lax
from jax.experimental import pallas as pl
from jax.experimental.pallas import tpu as pltpu
from jax.experimental.pallas import tpu_sc as plsc

BATCH = 4096
HIST = 50
NUM_ITEMS = 10
EMBED_DIM = 128

_INFO = plsc.get_sparse_core_info()
_NC = _INFO.num_cores
_NS = _INFO.num_subcores
_NW = _NC * _NS

_B_PER_W = BATCH // _NW
_CHUNK = _B_PER_W
_K = 5
_NWIN = HIST // _K


def _sc_lookup(x_blocks, table):
    mesh = plsc.VectorSubcoreMesh(core_axis_name="c", subcore_axis_name="s")

    @functools.partial(
        pl.kernel,
        mesh=mesh,
        out_type=jax.ShapeDtypeStruct((HIST, BATCH, EMBED_DIM), jnp.float32),
        scratch_types=[
            pltpu.VMEM_SHARED((_NS * NUM_ITEMS, EMBED_DIM), jnp.float32),
            pltpu.VMEM((HIST, _CHUNK), jnp.int32),
            pltpu.VMEM((_K, _CHUNK, EMBED_DIM), jnp.float32),
        ]
        + [pltpu.SemaphoreType.DMA] * (2 * _K),
    )
    def k(x_hbm, table_hbm, out_hbm, tab_sh, idx_v, bufs_v, *sems):
        gsem = sems[:_K]
        ssem = sems[_K:]
        sid = lax.axis_index("s")
        wid = sid * _NC + lax.axis_index("c")

        pltpu.sync_copy(table_hbm, tab_sh.at[pl.ds(sid * NUM_ITEMS, NUM_ITEMS)])
        pltpu.sync_copy(x_hbm.at[wid], idx_v)
        plsc.subcore_barrier()

        b0 = wid * _B_PER_W

        def window(g, carry):
            gh = []
            for b in range(_K):
                @pl.when(g > 0)
                def _(b=b):
                    pltpu.make_async_copy(
                        bufs_v.at[b], out_hbm.at[0, pl.ds(0, _CHUNK)], ssem[b]
                    ).wait()

                gh.append(
                    pltpu.async_copy(
                        tab_sh.at[idx_v.at[g * _K + b]], bufs_v.at[b], gsem[b]
                    )
                )
            for b in range(_K):
                gh[b].wait()
                pltpu.async_copy(
                    bufs_v.at[b],
                    out_hbm.at[g * _K + b, pl.ds(b0, _CHUNK)],
                    ssem[b],
                )
            return carry

        lax.fori_loop(0, _NWIN, window, 0, unroll=False)

        for b in range(_K):
            pltpu.make_async_copy(
                bufs_v.at[b], out_hbm.at[0, pl.ds(0, _CHUNK)], ssem[b]
            ).wait()

    return k(x_blocks, table)


def kernel(x, table):
    ids = x.astype(jnp.int32)
    ids = ids.T.reshape(HIST, _NW, _B_PER_W).transpose(1, 0, 2)
    sub = (jnp.arange(_NW, dtype=jnp.int32) // _NC) * NUM_ITEMS
    ids = ids + sub[:, None, None]
    out = _sc_lookup(ids, table)
    return out.transpose(1, 0, 2)
